# Initial kernel scaffold; baseline (speedup 1.0000x reference)
import jax
import jax.numpy as jnp
from jax.experimental import pallas as pl


def kernel(pointcloud, frozen, head):
    raise NotImplementedError("write your pallas kernel here")


# speedup vs baseline: 2.7334x; 2.7334x over previous
import functools

import jax
import jax.numpy as jnp
from jax.experimental import pallas as pl

_EPS = 1e-5
_HI = jax.lax.Precision.HIGHEST


def _fps_body(xyz_ref, new1_ref, new2_ref, new3_ref):
    def run(src, npoint, out_ref):
        B, N, _ = src.shape
        iota = jax.lax.broadcasted_iota(jnp.int32, (B, N), 1)

        def body(i, state):
            distance, farthest = state
            eq = (iota == farthest[:, None]).astype(src.dtype)
            c = jnp.sum(eq[:, :, None] * src, axis=1)
            out_ref[:, pl.ds(i, 1), :] = c[:, None, :]
            dist = jnp.sum((src - c[:, None, :]) ** 2, axis=-1)
            distance = jnp.minimum(distance, dist)
            m = jnp.max(distance, axis=-1, keepdims=True)
            far = jnp.min(jnp.where(distance == m, iota, N), axis=-1)
            return distance, far.astype(jnp.int32)

        state = (jnp.full((B, N), 1e10, dtype=src.dtype),
                 jnp.zeros((B,), jnp.int32))
        jax.lax.fori_loop(0, npoint, body, state)

    xyz = xyz_ref[...]
    run(xyz, new1_ref.shape[1], new1_ref)
    run(new1_ref[...], new2_ref.shape[1], new2_ref)
    run(new2_ref[...], new3_ref.shape[1], new3_ref)


def _shift_right(x, s):
    pad = jnp.zeros((x.shape[0], s), x.dtype)
    return jnp.concatenate([pad, x[:, : x.shape[1] - s]], axis=-1)


def _layer(x, wrefs, wi):
    W, b, gamma, beta, mean, var = (wrefs[wi + j][...] for j in range(6))
    x = jnp.dot(x.astype(jnp.bfloat16), W.astype(jnp.bfloat16),
                preferred_element_type=jnp.float32) + b
    x = (x - mean) / jnp.sqrt(var + _EPS) * gamma + beta
    return jnp.maximum(x, 0.0)


def _sa_body(*refs, radii_sq, Ks, n_layers, feat_ch, sb, n_src, head_mode):
    full_ref, new_ref = refs[0], refs[1]
    wrefs = refs[2:-1]
    out_ref = refs[-1]

    full = full_ref[0]
    new = new_ref[0]
    C = feat_ch
    xyz = full[:, C:C + 3]

    a2 = jnp.sum(new * new, axis=-1)[:, None]
    b2 = jnp.sum(xyz * xyz, axis=-1)[None, :]
    prod = jax.lax.dot_general(new.astype(jnp.bfloat16),
                               xyz.astype(jnp.bfloat16),
                               (((1,), (1,)), ((), ())),
                               preferred_element_type=jnp.float32)
    d = -2.0 * prod + a2 + b2

    wi = 0
    outs = []
    for bidx in range(len(Ks)):
        r2 = radii_sq[bidx]
        K = Ks[bidx]
        mask = d <= r2
        count0 = jnp.sum(mask.astype(jnp.int32), axis=-1, keepdims=True)
        is_last = jax.lax.broadcasted_iota(
            jnp.int32, (sb, n_src), 1) == (n_src - 1)
        mask = mask | ((count0 == 0) & is_last)
        mi = mask.astype(jnp.int32)
        inc = mi
        s = 1
        while s < n_src:
            inc = inc + _shift_right(inc, s)
            s *= 2
        rank = inc - mi
        count = inc[:, n_src - 1:n_src]
        kio = jax.lax.broadcasted_iota(jnp.int32, (sb, K), 1)
        kk = jnp.where(kio < count, kio, 0)
        oh = (mask[:, None, :] & (rank[:, None, :] == kk[:, :, None]))
        ohf = oh.astype(jnp.float32).reshape(sb * K, n_src)
        g = jnp.dot(ohf, full, precision=_HI,
                    preferred_element_type=jnp.float32)
        ctr = jnp.broadcast_to(new[:, None, :], (sb, K, 3)).reshape(sb * K, 3)
        x = jnp.concatenate([g[:, :C], g[:, C:] - ctr], axis=-1)
        for _ in range(n_layers):
            x = _layer(x, wrefs, wi)
            wi += 6
        outs.append(jnp.max(x.reshape(sb, K, x.shape[-1]), axis=1))

    cat = jnp.concatenate(outs, axis=-1)
    if head_mode:
        gmax = jnp.max(cat, axis=0, keepdims=True)
        W1, b1, gamma, beta, mean, var = (wrefs[wi + j][...] for j in range(6))
        y = jnp.dot(gmax.astype(jnp.bfloat16), W1.astype(jnp.bfloat16),
                    preferred_element_type=jnp.float32) + b1
        y = (y - mean) / jnp.sqrt(var + _EPS) * gamma + beta
        y = jnp.maximum(y, 0.0)
        W2, b2h = wrefs[wi + 6][...], wrefs[wi + 7][...]
        out_ref[0] = jnp.dot(y.astype(jnp.bfloat16), W2.astype(jnp.bfloat16),
                             preferred_element_type=jnp.float32) + b2h
    else:
        out_ref[0] = cat


def _layer_params(layer):
    return [layer['W'].T,
            layer['b'][None, :], layer['gamma'][None, :],
            layer['beta'][None, :], layer['mean'][None, :],
            layer['var'][None, :]]


def _sa_level(full, new, blocks, radii, Ks, sb, head_ws=None):
    B, n_src, Cf = full.shape
    C = Cf - 3
    S = new.shape[1]
    ws = []
    for blk in blocks:
        for layer in blk:
            ws += _layer_params(layer)
    if head_ws is not None:
        ws = ws + list(head_ws)
    ctot = blocks[0][-1]['W'].shape[0] + blocks[1][-1]['W'].shape[0]

    body = functools.partial(
        _sa_body, radii_sq=[r * r for r in radii], Ks=Ks,
        n_layers=len(blocks[0]), feat_ch=C, sb=sb, n_src=n_src,
        head_mode=head_ws is not None)

    w_specs = [pl.BlockSpec(w.shape, lambda b, s: (0, 0)) for w in ws]
    if head_ws is not None:
        grid = (B, 1)
        out_specs = pl.BlockSpec((1, 1, 248), lambda b, s: (b, 0, 0))
        out_shape = jax.ShapeDtypeStruct((B, 1, 248), jnp.float32)
    else:
        grid = (B, S // sb)
        out_specs = pl.BlockSpec((1, sb, ctot), lambda b, s: (b, s, 0))
        out_shape = jax.ShapeDtypeStruct((B, S, ctot), jnp.float32)

    in_specs = [
        pl.BlockSpec((1, n_src, Cf), lambda b, s: (b, 0, 0)),
        pl.BlockSpec((1, sb, 3), lambda b, s: (b, s, 0)),
    ] + w_specs

    return pl.pallas_call(
        body, grid=grid, in_specs=in_specs, out_specs=out_specs,
        out_shape=out_shape)(full, new, *ws)


def kernel(pointcloud, frozen, head):
    B, N0, _ = pointcloud.shape
    xyz0 = pointcloud[:, :, :3]

    new1, new2, new3 = pl.pallas_call(
        _fps_body,
        out_shape=[
            jax.ShapeDtypeStruct((B, 1024, 3), jnp.float32),
            jax.ShapeDtypeStruct((B, 256, 3), jnp.float32),
            jax.ShapeDtypeStruct((B, 64, 3), jnp.float32),
        ])(xyz0)

    full0 = jnp.concatenate([pointcloud, xyz0], axis=-1)
    l1 = _sa_level(full0, new1, frozen['sa1'], [0.05, 0.1], [16, 32], sb=8)

    full1 = jnp.concatenate([l1, new1], axis=-1)
    l2 = _sa_level(full1, new2, frozen['sa2'], [0.1, 0.2], [16, 32], sb=8)

    full2 = jnp.concatenate([l2, new2], axis=-1)
    head_ws = [
        head['fc1_W'].T, head['fc1_b'][None, :],
        head['bn1_gamma'][None, :], head['bn1_beta'][None, :],
        head['bn1_mean'][None, :], head['bn1_var'][None, :],
        head['fc2_W'].T, head['fc2_b'][None, :],
    ]
    out = _sa_level(full2, new3, frozen['sa3'], [0.2, 0.4], [16, 32],
                    sb=64, head_ws=head_ws)
    return out.reshape(B, 248)


# speedup vs baseline: 2.8274x; 1.0344x over previous
import functools

import jax
import jax.numpy as jnp
from jax.experimental import pallas as pl

_EPS = 1e-5
_HI = jax.lax.Precision.HIGHEST


def _fps_body(xyz_ref, new1_ref, new2_ref, new3_ref):
    def run(src, npoint, out_ref):
        B, N, _ = src.shape
        iota = jax.lax.broadcasted_iota(jnp.int32, (B, N), 1)

        def body(i, state):
            distance, farthest = state
            eq = (iota == farthest[:, None]).astype(src.dtype)
            c = jnp.sum(eq[:, :, None] * src, axis=1)
            out_ref[:, pl.ds(i, 1), :] = c[:, None, :]
            dist = jnp.sum((src - c[:, None, :]) ** 2, axis=-1)
            distance = jnp.minimum(distance, dist)
            m = jnp.max(distance, axis=-1, keepdims=True)
            far = jnp.min(jnp.where(distance == m, iota, N), axis=-1)
            return distance, far.astype(jnp.int32)

        state = (jnp.full((B, N), 1e10, dtype=src.dtype),
                 jnp.zeros((B,), jnp.int32))
        jax.lax.fori_loop(0, npoint, body, state)

    xyz = xyz_ref[...]
    run(xyz, new1_ref.shape[1], new1_ref)
    run(new1_ref[...], new2_ref.shape[1], new2_ref)
    run(new2_ref[...], new3_ref.shape[1], new3_ref)


def _shift_right(x, s):
    pad = jnp.zeros((x.shape[0], s), x.dtype)
    return jnp.concatenate([pad, x[:, : x.shape[1] - s]], axis=-1)


def _layer(x, wrefs, wi):
    W, b, gamma, beta, mean, var = (wrefs[wi + j][...] for j in range(6))
    x = jnp.dot(x.astype(jnp.bfloat16), W.astype(jnp.bfloat16),
                preferred_element_type=jnp.float32) + b
    x = (x - mean) / jnp.sqrt(var + _EPS) * gamma + beta
    return jnp.maximum(x, 0.0)


def _sa_body(*refs, radii_sq, Ks, n_layers, feat_ch, sb, n_src, head_mode):
    full_ref, new_ref = refs[0], refs[1]
    wrefs = refs[2:-1]
    out_ref = refs[-1]

    full = full_ref[0]
    new = new_ref[0]
    C = feat_ch
    xyz = full[:, C:C + 3]

    a2 = jnp.sum(new * new, axis=-1)[:, None]
    b2 = jnp.sum(xyz * xyz, axis=-1)[None, :]
    prod = jax.lax.dot_general(new.astype(jnp.bfloat16),
                               xyz.astype(jnp.bfloat16),
                               (((1,), (1,)), ((), ())),
                               preferred_element_type=jnp.float32)
    d = -2.0 * prod + a2 + b2

    wi = 0
    outs = []
    for bidx in range(len(Ks)):
        r2 = radii_sq[bidx]
        K = Ks[bidx]
        mask = d <= r2
        count0 = jnp.sum(mask.astype(jnp.int32), axis=-1, keepdims=True)
        is_last = jax.lax.broadcasted_iota(
            jnp.int32, (sb, n_src), 1) == (n_src - 1)
        mask = mask | ((count0 == 0) & is_last)
        mi = mask.astype(jnp.int32)
        inc = mi
        s = 1
        while s < n_src:
            inc = inc + _shift_right(inc, s)
            s *= 2
        rank = inc - mi
        count = inc[:, n_src - 1:n_src]
        kio = jax.lax.broadcasted_iota(jnp.int32, (sb, K), 1)
        kk = jnp.where(kio < count, kio, 0)
        oh = (mask[:, None, :] & (rank[:, None, :] == kk[:, :, None]))
        ohf = oh.astype(jnp.float32).reshape(sb * K, n_src)
        g = jnp.dot(ohf, full, precision=_HI,
                    preferred_element_type=jnp.float32)
        ctr = jnp.broadcast_to(new[:, None, :], (sb, K, 3)).reshape(sb * K, 3)
        x = jnp.concatenate([g[:, :C], g[:, C:] - ctr], axis=-1)
        for _ in range(n_layers):
            x = _layer(x, wrefs, wi)
            wi += 6
        outs.append(jnp.max(x.reshape(sb, K, x.shape[-1]), axis=1))

    cat = jnp.concatenate(outs, axis=-1)
    if head_mode:
        gmax = jnp.max(cat, axis=0, keepdims=True)
        W1, b1, gamma, beta, mean, var = (wrefs[wi + j][...] for j in range(6))
        y = jnp.dot(gmax.astype(jnp.bfloat16), W1.astype(jnp.bfloat16),
                    preferred_element_type=jnp.float32) + b1
        y = (y - mean) / jnp.sqrt(var + _EPS) * gamma + beta
        y = jnp.maximum(y, 0.0)
        W2, b2h = wrefs[wi + 6][...], wrefs[wi + 7][...]
        out_ref[0] = jnp.dot(y.astype(jnp.bfloat16), W2.astype(jnp.bfloat16),
                             preferred_element_type=jnp.float32) + b2h
    else:
        out_ref[0] = cat


def _layer_params(layer):
    return [layer['W'].T,
            layer['b'][None, :], layer['gamma'][None, :],
            layer['beta'][None, :], layer['mean'][None, :],
            layer['var'][None, :]]


def _sa_level(full, new, blocks, radii, Ks, sb, head_ws=None):
    B, n_src, Cf = full.shape
    C = Cf - 3
    S = new.shape[1]
    ws = []
    for blk in blocks:
        for layer in blk:
            ws += _layer_params(layer)
    if head_ws is not None:
        ws = ws + list(head_ws)
    ctot = blocks[0][-1]['W'].shape[0] + blocks[1][-1]['W'].shape[0]

    body = functools.partial(
        _sa_body, radii_sq=[r * r for r in radii], Ks=Ks,
        n_layers=len(blocks[0]), feat_ch=C, sb=sb, n_src=n_src,
        head_mode=head_ws is not None)

    w_specs = [pl.BlockSpec(w.shape, lambda b, s: (0, 0)) for w in ws]
    if head_ws is not None:
        grid = (B, 1)
        out_specs = pl.BlockSpec((1, 1, 248), lambda b, s: (b, 0, 0))
        out_shape = jax.ShapeDtypeStruct((B, 1, 248), jnp.float32)
    else:
        grid = (B, S // sb)
        out_specs = pl.BlockSpec((1, sb, ctot), lambda b, s: (b, s, 0))
        out_shape = jax.ShapeDtypeStruct((B, S, ctot), jnp.float32)

    in_specs = [
        pl.BlockSpec((1, n_src, Cf), lambda b, s: (b, 0, 0)),
        pl.BlockSpec((1, sb, 3), lambda b, s: (b, s, 0)),
    ] + w_specs

    return pl.pallas_call(
        body, grid=grid, in_specs=in_specs, out_specs=out_specs,
        out_shape=out_shape)(full, new, *ws)


def kernel(pointcloud, frozen, head):
    B, N0, _ = pointcloud.shape
    xyz0 = pointcloud[:, :, :3]

    new1, new2, new3 = pl.pallas_call(
        _fps_body,
        out_shape=[
            jax.ShapeDtypeStruct((B, 1024, 3), jnp.float32),
            jax.ShapeDtypeStruct((B, 256, 3), jnp.float32),
            jax.ShapeDtypeStruct((B, 64, 3), jnp.float32),
        ])(xyz0)

    full0 = jnp.concatenate([pointcloud, xyz0], axis=-1)
    l1 = _sa_level(full0, new1, frozen['sa1'], [0.05, 0.1], [16, 32], sb=32)

    full1 = jnp.concatenate([l1, new1], axis=-1)
    l2 = _sa_level(full1, new2, frozen['sa2'], [0.1, 0.2], [16, 32], sb=32)

    full2 = jnp.concatenate([l2, new2], axis=-1)
    head_ws = [
        head['fc1_W'].T, head['fc1_b'][None, :],
        head['bn1_gamma'][None, :], head['bn1_beta'][None, :],
        head['bn1_mean'][None, :], head['bn1_var'][None, :],
        head['fc2_W'].T, head['fc2_b'][None, :],
    ]
    out = _sa_level(full2, new3, frozen['sa3'], [0.2, 0.4], [16, 32],
                    sb=64, head_ws=head_ws)
    return out.reshape(B, 248)
